# Initial kernel scaffold; baseline (speedup 1.0000x reference)
import jax
import jax.numpy as jnp
from jax.experimental import pallas as pl


def kernel(student_logits, teacher_logits, labels):
    raise NotImplementedError("write your pallas kernel here")


# speedup vs baseline: 14.3015x; 14.3015x over previous
import functools

import jax
import jax.numpy as jnp
from jax.experimental import pallas as pl
from jax.experimental.pallas import tpu as pltpu

_TEMP = 5.0
_TOPK = 100
_IGNORE = -100
_LAMBDA = 10.0
_GAMMA = 1e-05
_NEG = -3.0e38
_I32_MIN = jnp.iinfo(jnp.int32).min


def _float_key(x):
    u = jax.lax.bitcast_convert_type(x, jnp.int32)
    return jnp.where(u >= 0, u, jnp.invert(u) + _I32_MIN)


def _loss_kernel(lab_ref, s_ref, t_ref, ce_ref, kl_ref, nv_ref, key_ref):
    i = pl.program_id(0)

    @pl.when(i == 0)
    def _init():
        ce_ref[...] = jnp.zeros((1, 1), jnp.float32)
        kl_ref[...] = jnp.zeros((1, 1), jnp.float32)
        nv_ref[...] = jnp.zeros((1, 1), jnp.float32)

    s = s_ref[...]
    t = t_ref[...]
    R, V = s.shape
    lab = lab_ref[0, 0, :]

    valid = lab != _IGNORE
    validf = valid.astype(jnp.float32)

    m_s = jnp.max(s, axis=1, keepdims=True)
    sumexp = jnp.sum(jnp.exp(s - m_s), axis=1)
    lse = jnp.log(sumexp) + m_s[:, 0]
    safe_lab = jnp.where(valid, lab, 0)
    col = jax.lax.broadcasted_iota(jnp.int32, (R, V), 1)
    onehot = col == safe_lab[:, None]
    s_lab = jnp.sum(jnp.where(onehot, s, 0.0), axis=1)
    ce_rows = (lse - s_lab) * validf

    keys = _float_key(t)
    key_ref[...] = keys
    hi0 = jnp.max(keys, axis=1, keepdims=True)
    lo0 = jnp.min(keys, axis=1, keepdims=True)

    def cond(carry):
        lo, hi = carry
        return jnp.any(lo < hi)

    def body(carry):
        lo, hi = carry
        mid = (lo & hi) + ((lo ^ hi) >> 1) + ((lo ^ hi) & 1)
        cnt = jnp.sum((key_ref[...] >= mid).astype(jnp.int32), axis=1,
                      keepdims=True)
        ge = cnt >= _TOPK
        lo = jnp.where(ge, mid, lo)
        hi = jnp.where(ge, hi, mid - 1)
        return lo, hi

    vkey, _ = jax.lax.while_loop(cond, body, (lo0, hi0))

    gt = keys > vkey
    eq = keys == vkey
    n_gt = jnp.sum(gt.astype(jnp.int32), axis=1, keepdims=True)
    n_eq = jnp.sum(eq.astype(jnp.int32), axis=1, keepdims=True)
    r = (_TOPK - n_gt).astype(jnp.float32)
    w_tie = r / n_eq.astype(jnp.float32)
    w = jnp.where(gt, 1.0, jnp.where(eq, w_tie, 0.0))

    t_max = jnp.max(t, axis=1, keepdims=True)
    et = w * jnp.exp((t - t_max) * (1.0 / _TEMP))
    a = jnp.sum(et, axis=1)
    b = jnp.sum(et * (t - s), axis=1) * (1.0 / _TEMP)
    selected = w > 0.0
    s_m = jnp.max(jnp.where(selected, s, _NEG), axis=1, keepdims=True)
    argc = jnp.where(selected, (s - s_m) * (1.0 / _TEMP), 0.0)
    c = jnp.sum(w * jnp.exp(argc), axis=1)
    kl_rows = (b / a - t_max[:, 0] * (1.0 / _TEMP) - jnp.log(a)
               + s_m[:, 0] * (1.0 / _TEMP) + jnp.log(c))
    kl_rows = kl_rows * validf

    ce_ref[...] += jnp.sum(ce_rows)[None, None]
    kl_ref[...] += jnp.sum(kl_rows)[None, None]
    nv_ref[...] += jnp.sum(validf)[None, None]


@jax.jit
def kernel(student_logits, teacher_logits, labels):
    B, N, V = student_logits.shape
    rows = B * N
    R = 32
    NB = rows // R
    s2 = student_logits.reshape(rows, V)
    t2 = teacher_logits.reshape(rows, V)
    lab3 = labels.reshape(NB, 1, R).astype(jnp.int32)

    out_shape = [jax.ShapeDtypeStruct((1, 1), jnp.float32)] * 3
    ce_sum, kl_sum, nv_sum = pl.pallas_call(
        _loss_kernel,
        grid=(NB,),
        in_specs=[
            pl.BlockSpec((1, 1, R), lambda i: (i, 0, 0)),
            pl.BlockSpec((R, V), lambda i: (i, 0)),
            pl.BlockSpec((R, V), lambda i: (i, 0)),
        ],
        out_specs=[pl.BlockSpec((1, 1), lambda i: (0, 0))] * 3,
        out_shape=out_shape,
        scratch_shapes=[pltpu.VMEM((R, V), jnp.int32)],
    )(lab3, s2, t2)

    nv = jnp.maximum(nv_sum[0, 0], 1.0)
    ce = ce_sum[0, 0] / nv
    tcs = kl_sum[0, 0] / nv * (_TEMP * _TEMP)
    attn = jnp.array(0.0, dtype=student_logits.dtype)
    total = ce + _LAMBDA * tcs + _GAMMA * attn
    return (total, ce, tcs, attn)
